# Initial kernel scaffold; baseline (speedup 1.0000x reference)
import jax
import jax.numpy as jnp
from jax.experimental import pallas as pl


def kernel(U, V, edge_u, edge_v, deg_u, deg_v, W1, W2):
    raise NotImplementedError("write your pallas kernel here")


# speedup vs baseline: 5.1018x; 5.1018x over previous
import functools

import jax
import jax.numpy as jnp
from jax import lax
from jax.experimental import pallas as pl
from jax.experimental.pallas import tpu as pltpu
from jax.experimental.pallas import tpu_sc as plsc

N = 10000
E = 320000
D = 128
LEAKY = 0.1

NS = 16
NG = 4
FG = D // NG
EPT = E // NS
CHUNK = 128
SUP = 1024
CPS = SUP // CHUNK
NSUP = -(-EPT // SUP)
TAIL = EPT - (NSUP - 1) * SUP


def _sc_body(ucat, vcat, eu_hbm, ev_hbm, du_hbm, dv_hbm,
             out_u, out_v,
             degu_v, degv_v, eu_s, ev_s, norm_s,
             euc_g, evc_g, euc_s, evc_s,
             urows, vrows, o_u, o_v,
             acc_u, acc_v, sem0, sem1):
    cid = lax.axis_index("c")
    sid = lax.axis_index("s")
    base = pl.multiple_of(sid * EPT, 8)

    pltpu.sync_copy(du_hbm, degu_v)
    pltpu.sync_copy(dv_hbm, degv_v)

    zi = jnp.zeros((16,), jnp.int32)
    zf = jnp.zeros((16,), jnp.float32)
    magic = jnp.full((16,), 0x5F3759DF, dtype=jnp.int32)
    one = jnp.full((16,), 1, dtype=jnp.int32)

    for p in range(2):
        g = p * 2 + cid
        goff = jnp.broadcast_to(jnp.int32(N) * g, (16,))

        def _zero_step(r, carry):
            for k in range(4):
                o_u[r, pl.ds(16 * k, 16)] = zf
                o_v[r, pl.ds(16 * k, 16)] = zf
            return carry

        lax.fori_loop(0, CHUNK, _zero_step, 0)

        @pl.when(sid < NS - 1)
        def _zero_main():
            for k in range(5):
                r0 = pl.multiple_of(sid * 640 + k * 128, 8)
                pltpu.sync_copy(o_u.at[pl.ds(0, 128)], acc_u.at[pl.ds(r0, 128)])
                pltpu.sync_copy(o_v.at[pl.ds(0, 128)], acc_v.at[pl.ds(r0, 128)])

        @pl.when(sid == NS - 1)
        def _zero_tail():
            for k in range(3):
                pltpu.sync_copy(o_u.at[pl.ds(0, 128)], acc_u.at[pl.ds(9600 + k * 128, 128)])
                pltpu.sync_copy(o_v.at[pl.ds(0, 128)], acc_v.at[pl.ds(9600 + k * 128, 128)])
            pltpu.sync_copy(o_u.at[pl.ds(0, 16)], acc_u.at[pl.ds(9984, 16)])
            pltpu.sync_copy(o_v.at[pl.ds(0, 16)], acc_v.at[pl.ds(9984, 16)])

        plsc.subcore_barrier()

        def _super_step(s, carry):
            sbase = base + s * SUP

            @pl.when(s < NSUP - 1)
            def _stage_full():
                pltpu.sync_copy(eu_hbm.at[pl.ds(sbase, SUP)], eu_s)
                pltpu.sync_copy(ev_hbm.at[pl.ds(sbase, SUP)], ev_s)

            @pl.when(s == NSUP - 1)
            def _stage_tail():
                pltpu.sync_copy(eu_hbm.at[pl.ds(sbase, TAIL)], eu_s.at[pl.ds(0, TAIL)])
                pltpu.sync_copy(ev_hbm.at[pl.ds(sbase, TAIL)], ev_s.at[pl.ds(0, TAIL)])
                for k in range((SUP - TAIL) // 16):
                    eu_s[pl.ds(TAIL + 16 * k, 16)] = zi
                    ev_s[pl.ds(TAIL + 16 * k, 16)] = zi

            def _norm_step(i, c2):
                off = i * 16
                eu16 = eu_s[pl.ds(off, 16)]
                ev16 = ev_s[pl.ds(off, 16)]
                du = plsc.load_gather(degu_v, [eu16])
                dv = plsc.load_gather(degv_v, [ev16])
                x = du * dv + 1e-8
                yi = magic - lax.shift_right_logical(plsc.bitcast(x, jnp.int32), one)
                y = plsc.bitcast(yi, jnp.float32)
                for _ in range(3):
                    y = y * (1.5 - 0.5 * x * y * y)
                norm_s[pl.ds(off, 16)] = y
                return c2

            lax.fori_loop(0, SUP // 16, _norm_step, 0)

            @pl.when(s == NSUP - 1)
            def _norm_tail():
                for k in range((SUP - TAIL) // 16):
                    norm_s[pl.ds(TAIL + 16 * k, 16)] = zf

            def _chunk_step(ci, c2):
                cb = ci * CHUNK
                for k in range(CHUNK // 16):
                    r_u = eu_s[pl.ds(cb + 16 * k, 16)]
                    r_v = ev_s[pl.ds(cb + 16 * k, 16)]
                    euc_s[pl.ds(16 * k, 16)] = r_u
                    evc_s[pl.ds(16 * k, 16)] = r_v
                    euc_g[pl.ds(16 * k, 16)] = r_u + goff
                    evc_g[pl.ds(16 * k, 16)] = r_v + goff
                cp_u = pltpu.async_copy(ucat.at[euc_g], urows, sem0)
                cp_v = pltpu.async_copy(vcat.at[evc_g], vrows, sem1)
                cp_u.wait()
                cp_v.wait()

                def _edge_step(e, c3):
                    nb = jnp.broadcast_to(norm_s[pl.ds(cb + e, 16)][0], (16,))
                    for h in range(FG // 16):
                        v16 = vrows[e, pl.ds(16 * h, 16)]
                        u16 = urows[e, pl.ds(16 * h, 16)]
                        nv = nb * v16
                        nu = nb * u16
                        npr = nv * u16
                        o_u[e, pl.ds(16 * h, 16)] = nv
                        o_u[e, pl.ds(FG + 16 * h, 16)] = npr
                        o_v[e, pl.ds(16 * h, 16)] = nu
                        o_v[e, pl.ds(FG + 16 * h, 16)] = npr
                    return c3

                lax.fori_loop(0, CHUNK, _edge_step, 0)
                pltpu.sync_copy(o_u, acc_u.at[euc_s], add=True)
                pltpu.sync_copy(o_v, acc_v.at[evc_s], add=True)
                return c2

            lax.fori_loop(0, CPS, _chunk_step, 0)
            return carry

        lax.fori_loop(0, NSUP, _super_step, 0)
        plsc.subcore_barrier()

        @pl.when(sid < NS - 1)
        def _write_main():
            rlo = pl.multiple_of(sid * 640, 8)
            pltpu.sync_copy(acc_u.at[pl.ds(rlo, 640)], out_u.at[g, pl.ds(rlo, 640)])
            pltpu.sync_copy(acc_v.at[pl.ds(rlo, 640)], out_v.at[g, pl.ds(rlo, 640)])

        @pl.when(sid == NS - 1)
        def _write_tail():
            pltpu.sync_copy(acc_u.at[pl.ds(9600, 400)], out_u.at[g, pl.ds(9600, 400)])
            pltpu.sync_copy(acc_v.at[pl.ds(9600, 400)], out_v.at[g, pl.ds(9600, 400)])

        if p == 0:
            plsc.subcore_barrier()


_sc_kernel = functools.partial(
    pl.kernel,
    out_type=(
        jax.ShapeDtypeStruct((NG, N, 2 * FG), jnp.float32),
        jax.ShapeDtypeStruct((NG, N, 2 * FG), jnp.float32),
    ),
    mesh=plsc.VectorSubcoreMesh(core_axis_name="c", subcore_axis_name="s"),
    compiler_params=pltpu.CompilerParams(
        needs_layout_passes=False, use_tc_tiling_on_sc=False),
    scratch_types=[
        pltpu.VMEM((N,), jnp.float32),
        pltpu.VMEM((N,), jnp.float32),
        pltpu.VMEM((SUP,), jnp.int32),
        pltpu.VMEM((SUP,), jnp.int32),
        pltpu.VMEM((SUP + 16,), jnp.float32),
        pltpu.VMEM((CHUNK,), jnp.int32),
        pltpu.VMEM((CHUNK,), jnp.int32),
        pltpu.VMEM((CHUNK,), jnp.int32),
        pltpu.VMEM((CHUNK,), jnp.int32),
        pltpu.VMEM((CHUNK, FG), jnp.float32),
        pltpu.VMEM((CHUNK, FG), jnp.float32),
        pltpu.VMEM((CHUNK, 2 * FG), jnp.float32),
        pltpu.VMEM((CHUNK, 2 * FG), jnp.float32),
        pltpu.VMEM_SHARED((N, 2 * FG), jnp.float32),
        pltpu.VMEM_SHARED((N, 2 * FG), jnp.float32),
        pltpu.SemaphoreType.DMA,
        pltpu.SemaphoreType.DMA,
    ],
)(_sc_body)


def _tc_body(u_ref, v_ref, cu_ref, cv_ref, w1_ref, w2_ref, un_ref, vn_ref):
    w1 = w1_ref[...]
    w2 = w2_ref[...]
    cu = cu_ref[...]
    cv = cv_ref[...]
    a_u = jnp.concatenate([cu[g, :, :FG] for g in range(NG)], axis=1)
    b_u = jnp.concatenate([cu[g, :, FG:] for g in range(NG)], axis=1)
    a_v = jnp.concatenate([cv[g, :, :FG] for g in range(NG)], axis=1)
    b_v = jnp.concatenate([cv[g, :, FG:] for g in range(NG)], axis=1)
    dn = (((1,), (1,)), ((), ()))
    yu = (lax.dot_general(u_ref[...] + a_u, w1, dn, preferred_element_type=jnp.float32)
          + lax.dot_general(b_u, w2, dn, preferred_element_type=jnp.float32))
    yv = (lax.dot_general(v_ref[...] + a_v, w1, dn, preferred_element_type=jnp.float32)
          + lax.dot_general(b_v, w2, dn, preferred_element_type=jnp.float32))
    un_ref[...] = jnp.where(yu >= 0, yu, LEAKY * yu)
    vn_ref[...] = jnp.where(yv >= 0, yv, LEAKY * yv)


_BLK = 1000

_tc_kernel = pl.pallas_call(
    _tc_body,
    grid=(N // _BLK,),
    in_specs=[
        pl.BlockSpec((_BLK, D), lambda i: (i, 0)),
        pl.BlockSpec((_BLK, D), lambda i: (i, 0)),
        pl.BlockSpec((NG, _BLK, 2 * FG), lambda i: (0, i, 0)),
        pl.BlockSpec((NG, _BLK, 2 * FG), lambda i: (0, i, 0)),
        pl.BlockSpec((D, D), lambda i: (0, 0)),
        pl.BlockSpec((D, D), lambda i: (0, 0)),
    ],
    out_specs=[
        pl.BlockSpec((_BLK, D), lambda i: (i, 0)),
        pl.BlockSpec((_BLK, D), lambda i: (i, 0)),
    ],
    out_shape=[
        jax.ShapeDtypeStruct((N, D), jnp.float32),
        jax.ShapeDtypeStruct((N, D), jnp.float32),
    ],
)


def kernel(U, V, edge_u, edge_v, deg_u, deg_v, W1, W2):
    ucat = U.reshape(N, NG, FG).transpose(1, 0, 2).reshape(NG * N, FG)
    vcat = V.reshape(N, NG, FG).transpose(1, 0, 2).reshape(NG * N, FG)
    c_u, c_v = _sc_kernel(ucat, vcat, edge_u, edge_v, deg_u, deg_v)
    u_new, v_new = _tc_kernel(U, V, c_u, c_v, W1, W2)
    return (u_new, v_new)


# speedup vs baseline: 10.5921x; 2.0761x over previous
import functools

import jax
import jax.numpy as jnp
from jax import lax
from jax.experimental import pallas as pl
from jax.experimental.pallas import tpu as pltpu
from jax.experimental.pallas import tpu_sc as plsc

N = 10000
E = 320000
D = 128
LEAKY = 0.1

NC = 2
NS = 16
NG = 4
FG = D // NG

EPT = E // NS
C = 128
CPS = 8
SUPE = C * CPS
NSUP = -(-EPT // SUPE)
TAILE = EPT - (NSUP - 1) * SUPE
NCH = NSUP * CPS

EPW = E // (NC * NS)
SUPP = 2000
NSUPP = EPW // SUPP

_SC_PARAMS = pltpu.CompilerParams(
    needs_layout_passes=False, use_tc_tiling_on_sc=False)
_SC_MESH = dict(core_axis_name="c", subcore_axis_name="s")


def _rsqrt16(x, magic, one):
    yi = magic - lax.shift_right_logical(plsc.bitcast(x, jnp.int32), one)
    y = plsc.bitcast(yi, jnp.float32)
    for _ in range(3):
        y = y * (1.5 - 0.5 * x * y * y)
    return y


def _norm_body(eu_hbm, ev_hbm, du_hbm, dv_hbm, nrm_out,
               degu_v, degv_v, eu_p, ev_p, nrm_p):
    cid = lax.axis_index("c")
    sid = lax.axis_index("s")
    wid = cid * NS + sid
    pbase = pl.multiple_of(wid * EPW, 8)

    pltpu.sync_copy(du_hbm, degu_v)
    pltpu.sync_copy(dv_hbm, degv_v)
    magic = jnp.full((16,), 0x5F3759DF, dtype=jnp.int32)
    one = jnp.full((16,), 1, dtype=jnp.int32)

    def _sup_step(s, carry):
        soff = pbase + s * SUPP
        pltpu.sync_copy(eu_hbm.at[pl.ds(soff, SUPP)], eu_p)
        pltpu.sync_copy(ev_hbm.at[pl.ds(soff, SUPP)], ev_p)

        def _norm_step(i, c2):
            off = i * 16
            du = plsc.load_gather(degu_v, [eu_p[pl.ds(off, 16)]])
            dv = plsc.load_gather(degv_v, [ev_p[pl.ds(off, 16)]])
            nrm_p[pl.ds(off, 16)] = _rsqrt16(du * dv + 1e-8, magic, one)
            return c2

        lax.fori_loop(0, SUPP // 16, _norm_step, 0)
        pltpu.sync_copy(nrm_p, nrm_out.at[pl.ds(soff, SUPP)])
        return carry

    lax.fori_loop(0, NSUPP, _sup_step, 0)


_norm_kernel = functools.partial(
    pl.kernel,
    out_type=jax.ShapeDtypeStruct((E,), jnp.float32),
    mesh=plsc.VectorSubcoreMesh(**_SC_MESH),
    compiler_params=_SC_PARAMS,
    scratch_types=[
        pltpu.VMEM((N,), jnp.float32),
        pltpu.VMEM((N,), jnp.float32),
        pltpu.VMEM((SUPP,), jnp.int32),
        pltpu.VMEM((SUPP,), jnp.int32),
        pltpu.VMEM((SUPP,), jnp.float32),
    ],
)(_norm_body)


def _main_body(ucat, vcat, eu_hbm, ev_hbm, nrm_hbm,
               out_u, out_v,
               eu_s, ev_s, norm_s,
               euc_g, evc_g, euc_s, evc_s,
               urows, vrows, o_au, o_av, o_p,
               acc_au, acc_bu, acc_av, acc_bv,
               g_sem, s_sem, t_sem):
    cid = lax.axis_index("c")
    sid = lax.axis_index("s")
    base = pl.multiple_of(sid * EPT, 8)
    zi = jnp.zeros((16,), jnp.int32)
    zf = jnp.zeros((16,), jnp.float32)

    def _stage_copies(snum, sp):
        soff = base + snum * SUPE
        if_tail = snum == NSUP - 1
        n_full = (eu_hbm.at[pl.ds(soff, SUPE)], eu_s.at[sp]), \
                 (ev_hbm.at[pl.ds(soff, SUPE)], ev_s.at[sp]), \
                 (nrm_hbm.at[pl.ds(soff, SUPE)], norm_s.at[sp, pl.ds(0, SUPE)])
        n_tail = (eu_hbm.at[pl.ds(soff, TAILE)], eu_s.at[sp, pl.ds(0, TAILE)]), \
                 (ev_hbm.at[pl.ds(soff, TAILE)], ev_s.at[sp, pl.ds(0, TAILE)]), \
                 (nrm_hbm.at[pl.ds(soff, TAILE)], norm_s.at[sp, pl.ds(0, TAILE)])
        return if_tail, n_full, n_tail

    def _stage_start(snum, sp):
        if_tail, n_full, n_tail = _stage_copies(snum, sp)

        @pl.when(jnp.logical_not(if_tail))
        def _full():
            for src, dst in n_full:
                pltpu.async_copy(src, dst, t_sem)

        @pl.when(if_tail)
        def _tail():
            for src, dst in n_tail:
                pltpu.async_copy(src, dst, t_sem)

    def _stage_wait(snum, sp):
        if_tail, n_full, n_tail = _stage_copies(snum, sp)

        @pl.when(jnp.logical_not(if_tail))
        def _full():
            for src, dst in n_full:
                pltpu.make_async_copy(src, dst, t_sem).wait()

        @pl.when(if_tail)
        def _tail():
            for src, dst in n_tail:
                pltpu.make_async_copy(src, dst, t_sem).wait()
            for k in range((SUPE - TAILE) // 16):
                eu_s[sp, pl.ds(TAILE + 16 * k, 16)] = zi
                ev_s[sp, pl.ds(TAILE + 16 * k, 16)] = zi
            for k in range((SUPE + 16 - TAILE) // 16):
                norm_s[sp, pl.ds(TAILE + 16 * k, 16)] = zf

    for p in range(2):
        g = p * 2 + cid
        goff = jnp.broadcast_to(jnp.int32(N) * g, (16,))

        def _prep_idx(cn, qn, rn):
            spn = (cn // CPS) & 1
            coff = (cn % CPS) * C
            for k in range(C // 16):
                r_u = eu_s[spn, pl.ds(coff + 16 * k, 16)]
                r_v = ev_s[spn, pl.ds(coff + 16 * k, 16)]
                euc_s[rn, pl.ds(16 * k, 16)] = r_u
                evc_s[rn, pl.ds(16 * k, 16)] = r_v
                euc_g[qn, pl.ds(16 * k, 16)] = r_u + goff
                evc_g[qn, pl.ds(16 * k, 16)] = r_v + goff

        def _zero_step(r, carry):
            for qq in range(2):
                for k in range(FG // 16):
                    o_au[qq, r, pl.ds(16 * k, 16)] = zf
                    o_av[qq, r, pl.ds(16 * k, 16)] = zf
                    o_p[qq, r, pl.ds(16 * k, 16)] = zf
            return carry

        lax.fori_loop(0, C, _zero_step, 0)
        for qq in (2, 3):
            for k in range(C // 16):
                euc_s[qq, pl.ds(16 * k, 16)] = zi
                evc_s[qq, pl.ds(16 * k, 16)] = zi

        for acc in (acc_au, acc_bu, acc_av, acc_bv):
            @pl.when(sid < NS - 1)
            def _zero_main(acc=acc):
                for k in range(5):
                    r0 = pl.multiple_of(sid * 640 + k * 128, 8)
                    pltpu.sync_copy(o_au.at[0], acc.at[pl.ds(r0, 128)])

            @pl.when(sid == NS - 1)
            def _zero_tail(acc=acc):
                for k in range(3):
                    pltpu.sync_copy(o_au.at[0], acc.at[pl.ds(9600 + k * 128, 128)])
                pltpu.sync_copy(o_au.at[0, pl.ds(0, 16)], acc.at[pl.ds(9984, 16)])

        plsc.subcore_barrier()

        for qq in range(2):
            pltpu.async_copy(o_au.at[qq], acc_au.at[euc_s.at[qq + 2]],
                             s_sem.at[qq], add=True)
            pltpu.async_copy(o_p.at[qq], acc_bu.at[euc_s.at[qq + 2]],
                             s_sem.at[qq], add=True)
            pltpu.async_copy(o_av.at[qq], acc_av.at[evc_s.at[qq + 2]],
                             s_sem.at[qq], add=True)
            pltpu.async_copy(o_p.at[qq], acc_bv.at[evc_s.at[qq + 2]],
                             s_sem.at[qq], add=True)

        _stage_start(jnp.int32(0), 0)
        _stage_wait(jnp.int32(0), 0)
        _prep_idx(jnp.int32(0), 0, 0)
        pltpu.async_copy(ucat.at[euc_g.at[0]], urows.at[0], g_sem.at[0])
        pltpu.async_copy(vcat.at[evc_g.at[0]], vrows.at[0], g_sem.at[0])

        def _chunk_step(c, carry):
            q = c & 1
            rd = (c + 2) & 3
            pltpu.make_async_copy(o_au.at[q], acc_au.at[euc_s.at[rd]],
                                  s_sem.at[q]).wait()
            pltpu.make_async_copy(o_p.at[q], acc_bu.at[euc_s.at[rd]],
                                  s_sem.at[q]).wait()
            pltpu.make_async_copy(o_av.at[q], acc_av.at[evc_s.at[rd]],
                                  s_sem.at[q]).wait()
            pltpu.make_async_copy(o_p.at[q], acc_bv.at[evc_s.at[rd]],
                                  s_sem.at[q]).wait()

            cn = c + 1

            @pl.when(cn < NCH)
            def _prefetch():
                qn = cn & 1
                cm = cn % CPS
                snum = cn // CPS

                @pl.when(cm == 0)
                def _stw():
                    _stage_wait(snum, snum & 1)

                @pl.when(jnp.logical_and(cm == 1, snum < NSUP - 1))
                def _sti():
                    _stage_start(snum + 1, (snum + 1) & 1)

                _prep_idx(cn, qn, cn & 3)
                pltpu.async_copy(ucat.at[euc_g.at[qn]], urows.at[qn],
                                 g_sem.at[qn])
                pltpu.async_copy(vcat.at[evc_g.at[qn]], vrows.at[qn],
                                 g_sem.at[qn])

            pltpu.make_async_copy(ucat.at[euc_g.at[q]], urows.at[q],
                                  g_sem.at[q]).wait()
            pltpu.make_async_copy(vcat.at[evc_g.at[q]], vrows.at[q],
                                  g_sem.at[q]).wait()

            sp_c = (c // CPS) & 1
            coff = (c % CPS) * C

            @plsc.parallel_loop(0, C // 16, unroll=2)
            def _edge_step(j):
                norm16 = norm_s[sp_c, pl.ds(coff + 16 * j, 16)]
                jb = 16 * j
                for e in range(16):
                    nb = jnp.broadcast_to(norm16[e], (16,))
                    for h in range(FG // 16):
                        v16 = vrows[q, jb + e, pl.ds(16 * h, 16)]
                        u16 = urows[q, jb + e, pl.ds(16 * h, 16)]
                        nv = nb * v16
                        nu = nb * u16
                        npr = nv * u16
                        o_au[q, jb + e, pl.ds(16 * h, 16)] = nv
                        o_av[q, jb + e, pl.ds(16 * h, 16)] = nu
                        o_p[q, jb + e, pl.ds(16 * h, 16)] = npr

            r = c & 3
            pltpu.async_copy(o_au.at[q], acc_au.at[euc_s.at[r]],
                             s_sem.at[q], add=True)
            pltpu.async_copy(o_p.at[q], acc_bu.at[euc_s.at[r]],
                             s_sem.at[q], add=True)
            pltpu.async_copy(o_av.at[q], acc_av.at[evc_s.at[r]],
                             s_sem.at[q], add=True)
            pltpu.async_copy(o_p.at[q], acc_bv.at[evc_s.at[r]],
                             s_sem.at[q], add=True)
            return carry

        lax.fori_loop(0, NCH, _chunk_step, 0)

        for qq in range(2):
            pltpu.make_async_copy(o_au.at[qq], acc_au.at[euc_s.at[qq]],
                                  s_sem.at[qq]).wait()
            pltpu.make_async_copy(o_p.at[qq], acc_bu.at[euc_s.at[qq]],
                                  s_sem.at[qq]).wait()
            pltpu.make_async_copy(o_av.at[qq], acc_av.at[evc_s.at[qq]],
                                  s_sem.at[qq]).wait()
            pltpu.make_async_copy(o_p.at[qq], acc_bv.at[evc_s.at[qq]],
                                  s_sem.at[qq]).wait()
        plsc.subcore_barrier()

        for ab, acc_x, acc_y in ((0, acc_au, acc_av), (1, acc_bu, acc_bv)):
            @pl.when(sid < NS - 1)
            def _write_main(ab=ab, acc_x=acc_x, acc_y=acc_y):
                rlo = pl.multiple_of(sid * 640, 8)
                pltpu.sync_copy(acc_x.at[pl.ds(rlo, 640)],
                                out_u.at[g, ab, pl.ds(rlo, 640)])
                pltpu.sync_copy(acc_y.at[pl.ds(rlo, 640)],
                                out_v.at[g, ab, pl.ds(rlo, 640)])

            @pl.when(sid == NS - 1)
            def _write_tail(ab=ab, acc_x=acc_x, acc_y=acc_y):
                pltpu.sync_copy(acc_x.at[pl.ds(9600, 400)],
                                out_u.at[g, ab, pl.ds(9600, 400)])
                pltpu.sync_copy(acc_y.at[pl.ds(9600, 400)],
                                out_v.at[g, ab, pl.ds(9600, 400)])

        if p == 0:
            plsc.subcore_barrier()


_main_kernel = functools.partial(
    pl.kernel,
    out_type=(
        jax.ShapeDtypeStruct((NG, 2, N, FG), jnp.float32),
        jax.ShapeDtypeStruct((NG, 2, N, FG), jnp.float32),
    ),
    mesh=plsc.VectorSubcoreMesh(**_SC_MESH),
    compiler_params=_SC_PARAMS,
    scratch_types=[
        pltpu.VMEM((2, SUPE), jnp.int32),
        pltpu.VMEM((2, SUPE), jnp.int32),
        pltpu.VMEM((2, SUPE + 16), jnp.float32),
        pltpu.VMEM((2, C), jnp.int32),
        pltpu.VMEM((2, C), jnp.int32),
        pltpu.VMEM((4, C), jnp.int32),
        pltpu.VMEM((4, C), jnp.int32),
        pltpu.VMEM((2, C, FG), jnp.float32),
        pltpu.VMEM((2, C, FG), jnp.float32),
        pltpu.VMEM((2, C, FG), jnp.float32),
        pltpu.VMEM((2, C, FG), jnp.float32),
        pltpu.VMEM((2, C, FG), jnp.float32),
        pltpu.VMEM_SHARED((N, FG), jnp.float32),
        pltpu.VMEM_SHARED((N, FG), jnp.float32),
        pltpu.VMEM_SHARED((N, FG), jnp.float32),
        pltpu.VMEM_SHARED((N, FG), jnp.float32),
        pltpu.SemaphoreType.DMA((2,)),
        pltpu.SemaphoreType.DMA((2,)),
        pltpu.SemaphoreType.DMA,
    ],
)(_main_body)


def _tc_body(u_ref, v_ref, cu_ref, cv_ref, w1_ref, w2_ref, un_ref, vn_ref):
    w1 = w1_ref[...]
    w2 = w2_ref[...]
    cu = cu_ref[...]
    cv = cv_ref[...]
    a_u = jnp.concatenate([cu[g, 0] for g in range(NG)], axis=1)
    b_u = jnp.concatenate([cu[g, 1] for g in range(NG)], axis=1)
    a_v = jnp.concatenate([cv[g, 0] for g in range(NG)], axis=1)
    b_v = jnp.concatenate([cv[g, 1] for g in range(NG)], axis=1)
    dn = (((1,), (1,)), ((), ()))
    yu = (lax.dot_general(u_ref[...] + a_u, w1, dn, preferred_element_type=jnp.float32)
          + lax.dot_general(b_u, w2, dn, preferred_element_type=jnp.float32))
    yv = (lax.dot_general(v_ref[...] + a_v, w1, dn, preferred_element_type=jnp.float32)
          + lax.dot_general(b_v, w2, dn, preferred_element_type=jnp.float32))
    un_ref[...] = jnp.where(yu >= 0, yu, LEAKY * yu)
    vn_ref[...] = jnp.where(yv >= 0, yv, LEAKY * yv)


_BLK = 1000

_tc_kernel = pl.pallas_call(
    _tc_body,
    grid=(N // _BLK,),
    in_specs=[
        pl.BlockSpec((_BLK, D), lambda i: (i, 0)),
        pl.BlockSpec((_BLK, D), lambda i: (i, 0)),
        pl.BlockSpec((NG, 2, _BLK, FG), lambda i: (0, 0, i, 0)),
        pl.BlockSpec((NG, 2, _BLK, FG), lambda i: (0, 0, i, 0)),
        pl.BlockSpec((D, D), lambda i: (0, 0)),
        pl.BlockSpec((D, D), lambda i: (0, 0)),
    ],
    out_specs=[
        pl.BlockSpec((_BLK, D), lambda i: (i, 0)),
        pl.BlockSpec((_BLK, D), lambda i: (i, 0)),
    ],
    out_shape=[
        jax.ShapeDtypeStruct((N, D), jnp.float32),
        jax.ShapeDtypeStruct((N, D), jnp.float32),
    ],
)


def kernel(U, V, edge_u, edge_v, deg_u, deg_v, W1, W2):
    ucat = U.reshape(N, NG, FG).transpose(1, 0, 2).reshape(NG * N, FG)
    vcat = V.reshape(N, NG, FG).transpose(1, 0, 2).reshape(NG * N, FG)
    nrm = _norm_kernel(edge_u, edge_v, deg_u, deg_v)
    c_u, c_v = _main_kernel(ucat, vcat, edge_u, edge_v, nrm)
    u_new, v_new = _tc_kernel(U, V, c_u, c_v, W1, W2)
    return (u_new, v_new)
